# baseline (speedup 1.0000x reference)
import jax
import jax.numpy as jnp
from jax import lax
from jax.experimental import pallas as pl
from jax.experimental.pallas import tpu as pltpu
from jax.experimental.pallas import tpu_sc as plsc

NNODES = 100000
NREL = 200
H = 16
E = 3200000

NC = 2
NS = 16
NW = NC * NS
SUB = 128
CHUNK = 256
NSUB = CHUNK // SUB
NCHUNKS = E // CHUNK
BASE_CH, EXTRA = divmod(NCHUNKS, NW)
NBLK = CHUNK // 16
TMAX = BASE_CH + 1
UNROLL = 6
NT2 = -(-TMAX // UNROLL)


SROWS = NNODES // NS


def _body(nodes_hbm, rel_hbm, idx_hbm, out_hbm,
          nodes_sp, rel_v, idx_v, so_rows, out_v,
          sem_i0, sem_i1, sem_i2, sem_g0, sem_g1, sem_o0, sem_o1):
    cid = lax.axis_index("c")
    sid = lax.axis_index("s")
    wid = sid * NC + cid
    nchunks = BASE_CH + jnp.where(wid < EXTRA, 1, 0)
    sem_i = (sem_i0, sem_i1, sem_i2)
    sem_g = (sem_g0, sem_g1)
    sem_o = (sem_o0, sem_o1)

    pltpu.sync_copy(nodes_hbm.at[pl.ds(sid * SROWS, SROWS)],
                    nodes_sp.at[pl.ds(sid * SROWS, SROWS)])
    pltpu.sync_copy(rel_hbm, rel_v)
    plsc.subcore_barrier()

    def fire_idx(t, p3):
        @pl.when(t < nchunks)
        def _():
            c = wid + NW * t
            pltpu.async_copy(idx_hbm.at[c], idx_v.at[p3], sem_i[p3])

    def wait_idx(t, p3):
        @pl.when(t < nchunks)
        def _():
            pltpu.make_async_copy(idx_hbm.at[0], idx_v.at[p3], sem_i[p3]).wait()

    def fire_gathers(t, p3, p2):
        @pl.when(t < nchunks)
        def _():
            pltpu.async_copy(nodes_sp.at[idx_v.at[p3, pl.ds(0, 2 * CHUNK)]],
                             so_rows.at[p2], sem_g[p2])

    def drain_gathers(t, p2):
        @pl.when(t < nchunks)
        def _():
            dummy = nodes_hbm.at[pl.ds(0, 2 * CHUNK)]
            pltpu.make_async_copy(dummy, so_rows.at[p2], sem_g[p2]).wait()

    def drain_out(t, p2):
        @pl.when(jnp.logical_and(t >= 0, t < nchunks))
        def _():
            pltpu.make_async_copy(out_hbm.at[pl.ds(0, CHUNK)],
                                  out_v.at[p2], sem_o[p2]).wait()

    def compute(t, p3, p2):
        @pl.when(t < nchunks)
        def _():
            c = wid + NW * t

            def blk(tb2, carry):
                for u in range(2):
                    tb = tb2 * 2 + u
                    rbase = tb * 16
                    iota = lax.iota(jnp.int32, 16)
                    rows = rbase + iota
                    p_ids = idx_v[p3, pl.ds(2 * CHUNK + tb * 16, 16)]
                    accs = [jnp.zeros(16, jnp.float32) for _ in range(4)]
                    for k in range(H):
                        hcol = jnp.bitwise_and(iota + k, H - 1)
                        sv = plsc.load_gather(so_rows.at[p2], [rows, hcol])
                        ov = plsc.load_gather(so_rows.at[p2],
                                              [rows + CHUNK, hcol])
                        pv = plsc.load_gather(rel_v, [p_ids, hcol])
                        accs[k % 4] = accs[k % 4] + sv * pv * ov
                    out_v[p2, pl.ds(rbase, 16)] = ((accs[0] + accs[1])
                                                   + (accs[2] + accs[3]))
                return carry

            lax.fori_loop(0, NBLK // 2, blk, 0)
            pltpu.async_copy(out_v.at[p2],
                             out_hbm.at[pl.ds(c * CHUNK, CHUNK)], sem_o[p2])

    fire_idx(0, 0)
    fire_idx(1, 1)
    wait_idx(0, 0)
    fire_gathers(0, 0, 0)

    def t2_body(t2, carry):
        tb0 = t2 * UNROLL
        for u in range(UNROLL):
            t = tb0 + u
            p3, p2 = u % 3, u % 2
            fire_idx(t + 2, (u + 2) % 3)
            wait_idx(t + 1, (u + 1) % 3)
            fire_gathers(t + 1, (u + 1) % 3, (u + 1) % 2)
            drain_gathers(t, p2)
            drain_out(t - 2, p2)
            compute(t, p3, p2)
        return carry

    lax.fori_loop(0, NT2, t2_body, 0)


@jax.jit
def kernel(nodes, relations, triples):
    s = triples[:, 0].reshape(NCHUNKS, CHUNK)
    p = triples[:, 1].reshape(NCHUNKS, CHUNK)
    o = triples[:, 2].reshape(NCHUNKS, CHUNK)
    idx = jnp.concatenate([s, o, p], axis=1)

    mesh = plsc.VectorSubcoreMesh(core_axis_name="c", subcore_axis_name="s")
    run = pl.kernel(
        _body,
        out_type=jax.ShapeDtypeStruct((E,), jnp.float32),
        mesh=mesh,
        compiler_params=pltpu.CompilerParams(needs_layout_passes=False,
                                             use_tc_tiling_on_sc=False),
        scratch_types=[
            pltpu.VMEM_SHARED((NNODES, H), jnp.float32),
            pltpu.VMEM((NREL, H), jnp.float32),
            pltpu.VMEM((3, 3 * CHUNK), jnp.int32),
            pltpu.VMEM((2, 2 * CHUNK, H), jnp.float32),
            pltpu.VMEM((2, CHUNK), jnp.float32),
            pltpu.SemaphoreType.DMA,
            pltpu.SemaphoreType.DMA,
            pltpu.SemaphoreType.DMA,
            pltpu.SemaphoreType.DMA,
            pltpu.SemaphoreType.DMA,
            pltpu.SemaphoreType.DMA,
            pltpu.SemaphoreType.DMA,
        ],
    )
    return run(nodes, relations, idx)


# speedup vs baseline: 1.4342x; 1.4342x over previous
import jax
import jax.numpy as jnp
from jax import lax
from jax.experimental import pallas as pl
from jax.experimental.pallas import tpu as pltpu
from jax.experimental.pallas import tpu_sc as plsc

NNODES = 100000
NREL = 200
H = 16
E = 3200000

NC = 2
NS = 16
NW = NC * NS
SUB = 128
CHUNK = 1024
NSUB = CHUNK // SUB
NCHUNKS = E // CHUNK
BASE_CH, EXTRA = divmod(NCHUNKS, NW)
NBLK = CHUNK // 16
TMAX = BASE_CH + 1
UNROLL = 6
NT2 = -(-TMAX // UNROLL)


def _body(nodes_hbm, rel_hbm, idx_hbm, out_hbm,
          rel_v, idx_v, s_rows, o_rows, out_v,
          sem_i0, sem_i1, sem_i2, sem_gs0, sem_gs1, sem_go0, sem_go1,
          sem_o0, sem_o1):
    cid = lax.axis_index("c")
    sid = lax.axis_index("s")
    wid = sid * NC + cid
    nchunks = BASE_CH + jnp.where(wid < EXTRA, 1, 0)
    sem_i = (sem_i0, sem_i1, sem_i2)
    sem_gs = (sem_gs0, sem_gs1)
    sem_go = (sem_go0, sem_go1)
    sem_o = (sem_o0, sem_o1)

    pltpu.sync_copy(rel_hbm, rel_v)

    def fire_idx(t, p3):
        @pl.when(t < nchunks)
        def _():
            c = wid + NW * t
            pltpu.async_copy(idx_hbm.at[c], idx_v.at[p3], sem_i[p3])

    def wait_idx(t, p3):
        @pl.when(t < nchunks)
        def _():
            pltpu.make_async_copy(idx_hbm.at[0], idx_v.at[p3], sem_i[p3]).wait()

    def fire_gathers(t, p3, p2):
        @pl.when(t < nchunks)
        def _():
            for j in range(NSUB):
                pltpu.async_copy(nodes_hbm.at[idx_v.at[p3, j]],
                                 s_rows.at[p2, pl.ds(j * SUB, SUB)], sem_gs[p2])
                pltpu.async_copy(nodes_hbm.at[idx_v.at[p3, NSUB + j]],
                                 o_rows.at[p2, pl.ds(j * SUB, SUB)], sem_go[p2])

    def drain_gathers(t, p2):
        @pl.when(t < nchunks)
        def _():
            dummy = nodes_hbm.at[pl.ds(0, CHUNK)]
            pltpu.make_async_copy(dummy, s_rows.at[p2], sem_gs[p2]).wait()
            pltpu.make_async_copy(dummy, o_rows.at[p2], sem_go[p2]).wait()

    def drain_out(t, p2):
        @pl.when(jnp.logical_and(t >= 0, t < nchunks))
        def _():
            pltpu.make_async_copy(out_hbm.at[pl.ds(0, CHUNK)],
                                  out_v.at[p2], sem_o[p2]).wait()

    def compute(t, p3, p2):
        @pl.when(t < nchunks)
        def _():
            c = wid + NW * t

            def blk(tb2, carry):
                for u in range(2):
                    tb = tb2 * 2 + u
                    rbase = tb * 16
                    iota = lax.iota(jnp.int32, 16)
                    rows = rbase + iota
                    p_ids = idx_v[p3, 2 * NSUB + tb // 8,
                                  pl.ds((tb % 8) * 16, 16)]
                    accs = [jnp.zeros(16, jnp.float32) for _ in range(4)]
                    for k in range(H):
                        hcol = jnp.bitwise_and(iota + k, H - 1)
                        sv = plsc.load_gather(s_rows.at[p2], [rows, hcol])
                        ov = plsc.load_gather(o_rows.at[p2], [rows, hcol])
                        pv = plsc.load_gather(rel_v, [p_ids, hcol])
                        accs[k % 4] = accs[k % 4] + sv * pv * ov
                    out_v[p2, pl.ds(rbase, 16)] = ((accs[0] + accs[1])
                                                   + (accs[2] + accs[3]))
                return carry

            lax.fori_loop(0, NBLK // 2, blk, 0)
            pltpu.async_copy(out_v.at[p2],
                             out_hbm.at[pl.ds(c * CHUNK, CHUNK)], sem_o[p2])

    fire_idx(0, 0)
    fire_idx(1, 1)
    wait_idx(0, 0)
    fire_gathers(0, 0, 0)

    def t2_body(t2, carry):
        tb0 = t2 * UNROLL
        for u in range(UNROLL):
            t = tb0 + u
            p3, p2 = u % 3, u % 2
            fire_idx(t + 2, (u + 2) % 3)
            wait_idx(t + 1, (u + 1) % 3)
            fire_gathers(t + 1, (u + 1) % 3, (u + 1) % 2)
            drain_gathers(t, p2)
            drain_out(t - 2, p2)
            compute(t, p3, p2)
        return carry

    lax.fori_loop(0, NT2, t2_body, 0)


@jax.jit
def kernel(nodes, relations, triples):
    s = triples[:, 0].reshape(NCHUNKS, NSUB, SUB)
    p = triples[:, 1].reshape(NCHUNKS, NSUB, SUB)
    o = triples[:, 2].reshape(NCHUNKS, NSUB, SUB)
    idx = jnp.concatenate([s, o, p], axis=1)

    mesh = plsc.VectorSubcoreMesh(core_axis_name="c", subcore_axis_name="s")
    run = pl.kernel(
        _body,
        out_type=jax.ShapeDtypeStruct((E,), jnp.float32),
        mesh=mesh,
        compiler_params=pltpu.CompilerParams(needs_layout_passes=False,
                                             use_tc_tiling_on_sc=False),
        scratch_types=[
            pltpu.VMEM((NREL, H), jnp.float32),
            pltpu.VMEM((3, 3 * NSUB, SUB), jnp.int32),
            pltpu.VMEM((2, CHUNK, H), jnp.float32),
            pltpu.VMEM((2, CHUNK, H), jnp.float32),
            pltpu.VMEM((2, CHUNK), jnp.float32),
            pltpu.SemaphoreType.DMA,
            pltpu.SemaphoreType.DMA,
            pltpu.SemaphoreType.DMA,
            pltpu.SemaphoreType.DMA,
            pltpu.SemaphoreType.DMA,
            pltpu.SemaphoreType.DMA,
            pltpu.SemaphoreType.DMA,
            pltpu.SemaphoreType.DMA,
            pltpu.SemaphoreType.DMA,
        ],
    )
    return run(nodes, relations, idx)
